# baseline (speedup 1.0000x reference)
import functools
import jax
import jax.numpy as jnp
from jax.experimental import pallas as pl
from jax.experimental.pallas import tpu as pltpu

_NW = 384
_NPAD = _NW * 32
_TB = 8
_RB = 256
_INTERPRET = False


def _row_map(which, k):
    def m(t, idx_ref):
        return (idx_ref[which, t * _TB + k], 0, 0)
    return m


def _gather_body(idx_ref, *refs):
    a_i = refs[0:_TB]
    a_j = refs[_TB:2 * _TB]
    x_i = refs[2 * _TB:3 * _TB]
    x_j = refs[3 * _TB:4 * _TB]
    cnw_ref = refs[4 * _TB]
    xij_ref = refs[4 * _TB + 1]

    ai = jnp.concatenate([r[0] for r in a_i], axis=0)
    aj = jnp.concatenate([r[0] for r in a_j], axis=0)
    cnw_ref[...] = ai & aj
    xi = jnp.concatenate([r[0] for r in x_i], axis=0)
    xj = jnp.concatenate([r[0] for r in x_j], axis=0)
    xij_ref[...] = xi * xj


def _mlp_body(cnw_ref, xij_ref, xg_ref, beta_ref,
              w1_ref, b1_ref, w2_ref, b2_ref, w3_ref, b3_ref,
              xw1_ref, xb1_ref, xw2_ref, xb2_ref,
              lw1_ref, lb1_ref, lw2_ref, lb2_ref, out_ref):
    f32 = jnp.float32
    w = cnw_ref[...]
    xcn = jnp.zeros((w.shape[0], xg_ref.shape[2]), f32)
    for b in range(32):
        mask = ((w >> b) & 1).astype(f32)
        xcn = xcn + jnp.dot(mask, xg_ref[b],
                            preferred_element_type=f32)
    h = jnp.maximum(jnp.dot(xcn, w1_ref[...], preferred_element_type=f32)
                    + b1_ref[...], 0.0)
    h = jnp.maximum(jnp.dot(h, w2_ref[...], preferred_element_type=f32)
                    + b2_ref[...], 0.0)
    h = jnp.dot(h, w3_ref[...], preferred_element_type=f32) + b3_ref[...]
    xij = xij_ref[...]
    g = jnp.maximum(jnp.dot(xij, xw1_ref[...], preferred_element_type=f32)
                    + xb1_ref[...], 0.0)
    g = jnp.dot(g, xw2_ref[...], preferred_element_type=f32) + xb2_ref[...]
    z = h * beta_ref[0, 0] + g
    z = jnp.maximum(jnp.dot(z, lw1_ref[...], preferred_element_type=f32)
                    + lb1_ref[...], 0.0)
    out_ref[...] = (jnp.dot(z, lw2_ref[...], preferred_element_type=f32)
                    + lb2_ref[...])


def kernel(x, edge_index, tar_ei, beta, xcn_w1, xcn_b1, xcn_w2, xcn_b2,
           xcn_w3, xcn_b3, xij_w1, xij_b1, xij_w2, xij_b2,
           lin_w1, lin_b1, lin_w2, lin_b2):
    n_nodes, in_ch = x.shape
    n_tar = tar_ei.shape[1]
    hid = xcn_w1.shape[1]
    out_ch = lin_w2.shape[1]

    key = edge_index[0] * 16384 + edge_index[1]
    sk = jnp.sort(key)
    first = jnp.concatenate([jnp.ones((1,), jnp.bool_), sk[1:] != sk[:-1]])
    v = sk & 16383
    widx = (sk >> 14) * _NW + (v >> 5)
    bit = jnp.where(first, jnp.left_shift(jnp.int32(1), v & 31), 0)
    adjw = jnp.zeros((n_nodes * _NW,), jnp.int32).at[widx].add(bit)
    adjw3 = adjw.reshape(n_nodes, 1, _NW)
    x3 = x.reshape(n_nodes, 1, in_ch)
    x_pad = jnp.pad(x, ((0, _NPAD - n_nodes), (0, 0)))
    xg = x_pad.reshape(_NW, 32, in_ch).transpose(1, 0, 2)
    idx = tar_ei.astype(jnp.int32)

    n_steps = n_tar // _TB
    in_specs = (
        [pl.BlockSpec((1, 1, _NW), _row_map(0, k)) for k in range(_TB)]
        + [pl.BlockSpec((1, 1, _NW), _row_map(1, k)) for k in range(_TB)]
        + [pl.BlockSpec((1, 1, in_ch), _row_map(0, k)) for k in range(_TB)]
        + [pl.BlockSpec((1, 1, in_ch), _row_map(1, k)) for k in range(_TB)]
    )
    out_specs = [
        pl.BlockSpec((_TB, _NW), lambda t, idx_ref: (t, 0)),
        pl.BlockSpec((_TB, in_ch), lambda t, idx_ref: (t, 0)),
    ]
    grid_spec = pltpu.PrefetchScalarGridSpec(
        num_scalar_prefetch=1,
        grid=(n_steps,),
        in_specs=in_specs,
        out_specs=out_specs,
    )
    cnw, xij = pl.pallas_call(
        _gather_body,
        grid_spec=grid_spec,
        out_shape=[
            jax.ShapeDtypeStruct((n_tar, _NW), jnp.int32),
            jax.ShapeDtypeStruct((n_tar, in_ch), jnp.float32),
        ],
        compiler_params=pltpu.CompilerParams(
            dimension_semantics=("arbitrary",),
        ),
        interpret=_INTERPRET,
    )(idx, *([adjw3] * (2 * _TB)), *([x3] * (2 * _TB)))

    full = lambda shape: pl.BlockSpec(shape, lambda r: (0,) * len(shape))
    out = pl.pallas_call(
        _mlp_body,
        grid=(n_tar // _RB,),
        in_specs=[
            pl.BlockSpec((_RB, _NW), lambda r: (r, 0)),
            pl.BlockSpec((_RB, in_ch), lambda r: (r, 0)),
            full((32, _NW, in_ch)),
            full((1, 1)),
            full((in_ch, hid)), full((1, hid)),
            full((hid, hid)), full((1, hid)),
            full((hid, hid)), full((1, hid)),
            full((in_ch, hid)), full((1, hid)),
            full((hid, hid)), full((1, hid)),
            full((hid, hid)), full((1, hid)),
            full((hid, out_ch)), full((1, out_ch)),
        ],
        out_specs=pl.BlockSpec((_RB, out_ch), lambda r: (r, 0)),
        out_shape=jax.ShapeDtypeStruct((n_tar, out_ch), jnp.float32),
        compiler_params=pltpu.CompilerParams(
            dimension_semantics=("arbitrary",),
        ),
        interpret=_INTERPRET,
    )(cnw, xij, xg, beta.reshape(1, 1),
      xcn_w1, xcn_b1.reshape(1, hid), xcn_w2, xcn_b2.reshape(1, hid),
      xcn_w3, xcn_b3.reshape(1, hid),
      xij_w1, xij_b1.reshape(1, hid), xij_w2, xij_b2.reshape(1, hid),
      lin_w1, lin_b1.reshape(1, hid), lin_w2, lin_b2.reshape(1, out_ch))
    return out


# speedup vs baseline: 1.5760x; 1.5760x over previous
import functools
import jax
import jax.numpy as jnp
from jax import lax
from jax.experimental import pallas as pl
from jax.experimental.pallas import tpu as pltpu
from jax.experimental.pallas import tpu_sc as plsc

_NW = 384
_NPAD = _NW * 32
_TB = 8
_RB = 256
_INTERPRET = False


def _sc_gather(adjw, idx0, idx1, x):
    n_tar = idx0.shape[0]
    in_ch = x.shape[1]
    info = plsc.get_sparse_core_info()
    nc, ns = info.num_cores, info.num_subcores
    nworkers = nc * ns
    per_w = n_tar // nworkers
    ch = min(64, per_w)
    n_ch = per_w // ch
    mesh = plsc.VectorSubcoreMesh(core_axis_name="c", subcore_axis_name="s")

    @functools.partial(
        pl.kernel, mesh=mesh,
        out_type=[
            jax.ShapeDtypeStruct((n_tar, _NW), jnp.int32),
            jax.ShapeDtypeStruct((n_tar, in_ch), jnp.float32),
        ],
        scratch_types=[
            pltpu.VMEM((ch,), jnp.int32),
            pltpu.VMEM((ch,), jnp.int32),
            pltpu.VMEM((ch, _NW), jnp.int32),
            pltpu.VMEM((ch, _NW), jnp.int32),
            pltpu.VMEM((ch, in_ch), jnp.float32),
            pltpu.VMEM((ch, in_ch), jnp.float32),
            pltpu.SemaphoreType.DMA,
        ],
    )
    def k(adjw_hbm, idx0_hbm, idx1_hbm, x_hbm, cnw_hbm, xij_hbm,
          ii_v, ij_v, ai_v, aj_v, xi_v, xj_v, sem):
        wid = lax.axis_index("s") * nc + lax.axis_index("c")
        for c in range(n_ch):
            base = wid * per_w + c * ch
            pltpu.sync_copy(idx0_hbm.at[pl.ds(base, ch)], ii_v)
            pltpu.sync_copy(idx1_hbm.at[pl.ds(base, ch)], ij_v)
            pltpu.async_copy(adjw_hbm.at[ii_v], ai_v, sem).wait()
            pltpu.async_copy(adjw_hbm.at[ij_v], aj_v, sem).wait()
            pltpu.async_copy(x_hbm.at[ii_v], xi_v, sem).wait()
            pltpu.async_copy(x_hbm.at[ij_v], xj_v, sem).wait()

            nwv = _NW // 16

            def and_body(t, carry):
                r = t // nwv
                o = (t % nwv) * 16
                ai_v[r, pl.ds(o, 16)] = (ai_v[r, pl.ds(o, 16)]
                                         & aj_v[r, pl.ds(o, 16)])
                return carry

            lax.fori_loop(0, ch * nwv, and_body, 0)

            icv = in_ch // 16

            def mul_body(t, carry):
                r = t // icv
                o = (t % icv) * 16
                xi_v[r, pl.ds(o, 16)] = (xi_v[r, pl.ds(o, 16)]
                                         * xj_v[r, pl.ds(o, 16)])
                return carry

            lax.fori_loop(0, ch * icv, mul_body, 0)

            pltpu.sync_copy(ai_v, cnw_hbm.at[pl.ds(base, ch)])
            pltpu.sync_copy(xi_v, xij_hbm.at[pl.ds(base, ch)])

    return k(adjw, idx0, idx1, x)


def _mlp_body(cnw_ref, xij_ref, xg_ref, beta_ref,
              w1_ref, b1_ref, w2_ref, b2_ref, w3_ref, b3_ref,
              xw1_ref, xb1_ref, xw2_ref, xb2_ref,
              lw1_ref, lb1_ref, lw2_ref, lb2_ref, out_ref):
    f32 = jnp.float32
    w = cnw_ref[...]
    xcn = jnp.zeros((w.shape[0], xg_ref.shape[2]), f32)
    for b in range(32):
        mask = ((w >> b) & 1).astype(f32)
        xcn = xcn + jnp.dot(mask, xg_ref[b],
                            preferred_element_type=f32)
    h = jnp.maximum(jnp.dot(xcn, w1_ref[...], preferred_element_type=f32)
                    + b1_ref[...], 0.0)
    h = jnp.maximum(jnp.dot(h, w2_ref[...], preferred_element_type=f32)
                    + b2_ref[...], 0.0)
    h = jnp.dot(h, w3_ref[...], preferred_element_type=f32) + b3_ref[...]
    xij = xij_ref[...]
    g = jnp.maximum(jnp.dot(xij, xw1_ref[...], preferred_element_type=f32)
                    + xb1_ref[...], 0.0)
    g = jnp.dot(g, xw2_ref[...], preferred_element_type=f32) + xb2_ref[...]
    z = h * beta_ref[0, 0] + g
    z = jnp.maximum(jnp.dot(z, lw1_ref[...], preferred_element_type=f32)
                    + lb1_ref[...], 0.0)
    out_ref[...] = (jnp.dot(z, lw2_ref[...], preferred_element_type=f32)
                    + lb2_ref[...])


def kernel(x, edge_index, tar_ei, beta, xcn_w1, xcn_b1, xcn_w2, xcn_b2,
           xcn_w3, xcn_b3, xij_w1, xij_b1, xij_w2, xij_b2,
           lin_w1, lin_b1, lin_w2, lin_b2):
    n_nodes, in_ch = x.shape
    n_tar = tar_ei.shape[1]
    hid = xcn_w1.shape[1]
    out_ch = lin_w2.shape[1]

    key = edge_index[0] * 16384 + edge_index[1]
    sk = jnp.sort(key)
    first = jnp.concatenate([jnp.ones((1,), jnp.bool_), sk[1:] != sk[:-1]])
    v = sk & 16383
    widx = (sk >> 14) * _NW + (v >> 5)
    bit = jnp.where(first, jnp.left_shift(jnp.int32(1), v & 31), 0)
    adjw = jnp.zeros((n_nodes * _NW,), jnp.int32).at[widx].add(bit)
    x_pad = jnp.pad(x, ((0, _NPAD - n_nodes), (0, 0)))
    xg = x_pad.reshape(_NW, 32, in_ch).transpose(1, 0, 2)
    idx = tar_ei.astype(jnp.int32)

    cnw, xij = _sc_gather(adjw.reshape(n_nodes, _NW), idx[0], idx[1], x)

    full = lambda shape: pl.BlockSpec(shape, lambda r: (0,) * len(shape))
    out = pl.pallas_call(
        _mlp_body,
        grid=(n_tar // _RB,),
        in_specs=[
            pl.BlockSpec((_RB, _NW), lambda r: (r, 0)),
            pl.BlockSpec((_RB, in_ch), lambda r: (r, 0)),
            full((32, _NW, in_ch)),
            full((1, 1)),
            full((in_ch, hid)), full((1, hid)),
            full((hid, hid)), full((1, hid)),
            full((hid, hid)), full((1, hid)),
            full((in_ch, hid)), full((1, hid)),
            full((hid, hid)), full((1, hid)),
            full((hid, hid)), full((1, hid)),
            full((hid, out_ch)), full((1, out_ch)),
        ],
        out_specs=pl.BlockSpec((_RB, out_ch), lambda r: (r, 0)),
        out_shape=jax.ShapeDtypeStruct((n_tar, out_ch), jnp.float32),
        compiler_params=pltpu.CompilerParams(
            dimension_semantics=("arbitrary",),
        ),
        interpret=_INTERPRET,
    )(cnw, xij, xg, beta.reshape(1, 1),
      xcn_w1, xcn_b1.reshape(1, hid), xcn_w2, xcn_b2.reshape(1, hid),
      xcn_w3, xcn_b3.reshape(1, hid),
      xij_w1, xij_b1.reshape(1, hid), xij_w2, xij_b2.reshape(1, hid),
      lin_w1, lin_b1.reshape(1, hid), lin_w2, lin_b2.reshape(1, out_ch))
    return out
